# baseline (speedup 1.0000x reference)
import dataclasses
import functools

import jax
import jax.numpy as jnp
from jax import lax
from jax.experimental import pallas as pl
from jax.experimental.pallas import tpu as pltpu
from jax.experimental.pallas import tpu_sc as plsc

N = 10000
E = 320000
NS = 16
K = 128
C16 = 160
C32 = 80
CB = 32
CB0 = 40
NPAD = N + 16
STRIPE = 624
TAIL_A = NPAD - NS * STRIPE
TAIL_O = N - NS * STRIPE
N_STEP = 10

_mesh = plsc.VectorSubcoreMesh(core_axis_name="c", subcore_axis_name="s")
_HIGH = jax.lax.Precision.HIGHEST


def _f32(shape):
    return jax.ShapeDtypeStruct(shape, jnp.float32)


def _chunked(idx, workers, chunks, pad_vals):
    per = E // workers
    padw = chunks * K - per
    return jnp.concatenate(
        [idx.reshape(workers, per), jnp.broadcast_to(pad_vals[:padw], (workers, padw))],
        axis=1,
    ).reshape(workers * chunks, K)


def _chunk_edges(src, dst):
    pad_real = (jnp.arange(512, dtype=jnp.int32) * 37) % N
    pad_dummy = N + (jnp.arange(512, dtype=jnp.int32) % 16)
    sg16 = _chunked(src, NS, C16, pad_real)
    dg16 = _chunked(dst, NS, C16, pad_dummy)
    sd16 = _chunked(src, NS, C16, pad_dummy)
    sg32 = _chunked(src, 2 * NS, C32, pad_real)
    dg32 = _chunked(dst, 2 * NS, C32, pad_dummy)
    return sg16, dg16, sd16, sg32, dg32


def _agg_pipeline(h_h, sg_h, dg_h, acc, sidx_v, didx_v, rows_a, rows_b,
                  sem_a, sem_b, base0, nblk, cb):
    @pl.loop(0, nblk)
    def _(b):
        base = base0 + b * cb
        pltpu.sync_copy(sg_h.at[pl.ds(base, cb)], sidx_v)
        pltpu.sync_copy(dg_h.at[pl.ds(base, cb)], didx_v)
        pltpu.async_copy(h_h.at[sidx_v.at[0]], rows_a, sem_a)

        @pl.loop(0, cb, step=2)
        def _(j):
            pltpu.make_async_copy(h_h.at[sidx_v.at[j]], rows_a, sem_a).wait()
            pltpu.async_copy(h_h.at[sidx_v.at[j + 1]], rows_b, sem_b)
            pltpu.sync_copy(rows_a, acc.at[didx_v.at[j]], add=True)
            pltpu.make_async_copy(h_h.at[sidx_v.at[j + 1]], rows_b, sem_b).wait()

            @pl.when(j + 2 < cb)
            def _():
                pltpu.async_copy(h_h.at[sidx_v.at[j + 2]], rows_a, sem_a)

            pltpu.sync_copy(rows_b, acc.at[didx_v.at[j + 1]], add=True)

EWP = C16 * K
DH = 128

_sc_cp = pltpu.CompilerParams()
if "needs_layout_passes" in pltpu.CompilerParams.__dataclass_fields__:
    _sc_cp = dataclasses.replace(_sc_cp, needs_layout_passes=False)


@functools.partial(
    pl.kernel,
    out_type=[_f32((DH, 128)), _f32((DH, 128))],
    mesh=_mesh,
    compiler_params=_sc_cp,
    scratch_types=[
        pltpu.VMEM((EWP,), jnp.int32),
        pltpu.VMEM((DH, 128), jnp.float32),
        pltpu.VMEM((1, 128), jnp.int32),
        pltpu.VMEM_SHARED((DH, 128), jnp.float32),
    ],
)
def _sc_degrees(sdf_h, dgf_h, iota_h, zeros_h, dego_h, degi_h,
                idx_v, hist, idr_v, acc):
    cid = lax.axis_index("c")
    wid = lax.axis_index("s")
    pltpu.sync_copy(iota_h, idr_v)
    pltpu.sync_copy(zeros_h.at[pl.ds(0, DH)], hist)

    @pl.when(wid == 0)
    def _():
        pltpu.sync_copy(zeros_h.at[pl.ds(0, DH)], acc)

    ones16 = jnp.full((16,), 1.0, jnp.float32)

    def run(idxf_h, out_h):
        pltpu.sync_copy(idxf_h.at[pl.ds(wid * EWP, EWP)], idx_v)
        plsc.subcore_barrier()

        @pl.loop(0, EWP, step=16)
        def _(i):
            v = idx_v[pl.ds(i, 16)]
            r = lax.shift_right_logical(v, 7)
            c = lax.bitwise_and(v, 127)
            plsc.addupdate_scatter(hist, [r, c], ones16)

        pltpu.sync_copy(hist, acc.at[idr_v.at[0]], add=True)
        plsc.subcore_barrier()

        @pl.when(wid == 0)
        def _():
            pltpu.sync_copy(acc, out_h)

    @pl.when(cid == 0)
    def _():
        run(sdf_h, dego_h)

    @pl.when(cid == 1)
    def _():
        run(dgf_h, degi_h)


@functools.partial(
    pl.kernel,
    out_type=[_f32((N, 128)), _f32((N, 128))],
    mesh=_mesh,
    scratch_types=[
        pltpu.VMEM((CB0, K), jnp.int32),
        pltpu.VMEM((CB0, K), jnp.int32),
        pltpu.VMEM((K, 128), jnp.float32),
        pltpu.VMEM((K, 128), jnp.float32),
        pltpu.VMEM_SHARED((NPAD, 128), jnp.float32),
        pltpu.SemaphoreType.DMA,
        pltpu.SemaphoreType.DMA,
    ],
)
def _sc_aggregate_l0(xn_h, sg32_h, dg32_h, zeros_h, out0_h, out1_h,
                     sidx_v, didx_v, rows_a, rows_b, acc, sem_a, sem_b):
    cid = lax.axis_index("c")
    wid = lax.axis_index("s")
    gwid = cid * NS + wid
    pltpu.sync_copy(zeros_h.at[pl.ds(wid * STRIPE, STRIPE)],
                    acc.at[pl.ds(wid * STRIPE, STRIPE)])

    @pl.when(wid == 0)
    def _():
        pltpu.sync_copy(zeros_h.at[pl.ds(NS * STRIPE, TAIL_A)],
                        acc.at[pl.ds(NS * STRIPE, TAIL_A)])

    plsc.subcore_barrier()

    _agg_pipeline(xn_h, sg32_h, dg32_h, acc, sidx_v, didx_v, rows_a, rows_b,
                  sem_a, sem_b, gwid * C32, C32 // CB0, CB0)

    plsc.subcore_barrier()

    def flush(out_h):
        pltpu.sync_copy(acc.at[pl.ds(wid * STRIPE, STRIPE)],
                        out_h.at[pl.ds(wid * STRIPE, STRIPE)])

        @pl.when(wid == 0)
        def _():
            pltpu.sync_copy(acc.at[pl.ds(NS * STRIPE, TAIL_O)],
                            out_h.at[pl.ds(NS * STRIPE, TAIL_O)])

    @pl.when(cid == 0)
    def _():
        flush(out0_h)

    @pl.when(cid == 1)
    def _():
        flush(out1_h)


@functools.partial(
    pl.kernel,
    out_type=[_f32((N, 128)), _f32((N, 128))],
    mesh=_mesh,
    scratch_types=[
        pltpu.VMEM((CB, K), jnp.int32),
        pltpu.VMEM((CB, K), jnp.int32),
        pltpu.VMEM((K, 128), jnp.float32),
        pltpu.VMEM((K, 128), jnp.float32),
        pltpu.VMEM_SHARED((NPAD, 128), jnp.float32),
        pltpu.SemaphoreType.DMA,
        pltpu.SemaphoreType.DMA,
    ],
)
def _sc_aggregate(h0_h, h1_h, sg16_h, dg16_h, zeros_h, out0_h, out1_h,
                  sidx_v, didx_v, rows_a, rows_b, acc, sem_a, sem_b):
    cid = lax.axis_index("c")
    wid = lax.axis_index("s")
    pltpu.sync_copy(zeros_h.at[pl.ds(wid * STRIPE, STRIPE)],
                    acc.at[pl.ds(wid * STRIPE, STRIPE)])

    @pl.when(wid == 0)
    def _():
        pltpu.sync_copy(zeros_h.at[pl.ds(NS * STRIPE, TAIL_A)],
                        acc.at[pl.ds(NS * STRIPE, TAIL_A)])

    plsc.subcore_barrier()

    def run(h_h, out_h):
        _agg_pipeline(h_h, sg16_h, dg16_h, acc, sidx_v, didx_v, rows_a,
                      rows_b, sem_a, sem_b, wid * C16, C16 // CB, CB)
        plsc.subcore_barrier()
        pltpu.sync_copy(acc.at[pl.ds(wid * STRIPE, STRIPE)],
                        out_h.at[pl.ds(wid * STRIPE, STRIPE)])

        @pl.when(wid == 0)
        def _():
            pltpu.sync_copy(acc.at[pl.ds(NS * STRIPE, TAIL_O)],
                            out_h.at[pl.ds(NS * STRIPE, TAIL_O)])

    @pl.when(cid == 0)
    def _():
        run(h0_h, out0_h)

    @pl.when(cid == 1)
    def _():
        run(h1_h, out1_h)


def _inv_sqrt_deg(deg_col):
    return jnp.where(deg_col > 0.0, lax.rsqrt(jnp.maximum(deg_col, 1e-30)), 0.0)


def _leaky(h):
    return jnp.where(h > 0.0, h, 0.01 * h)


def _norm_x_kernel(x_ref, dego_ref, xn_ref):
    ns = _inv_sqrt_deg(dego_ref[...])
    xn_ref[...] = x_ref[...] * ns


def _norm_x(x, dego):
    rb = 1000
    return pl.pallas_call(
        _norm_x_kernel,
        grid=(N // rb,),
        in_specs=[
            pl.BlockSpec((rb, 128), lambda i: (i, 0)),
            pl.BlockSpec((rb, 1), lambda i: (i, 0)),
        ],
        out_specs=pl.BlockSpec((rb, 128), lambda i: (i, 0)),
        out_shape=_f32((N, 128)),
    )(x, dego)


def _layer_kernel(first, last, a0_ref, a1_ref, W_ref, b_ref, degi_ref,
                  dego_ref, Wn_ref, feat_ref, h0_ref, h1_ref):
    nd = _inv_sqrt_deg(degi_ref[...])
    if first:
        aggv = a0_ref[...] + a1_ref[...]
        h = lax.dot_general(aggv, W_ref[...], (((1,), (0,)), ((), ())),
                            precision=_HIGH)
    else:
        h = jnp.concatenate([a0_ref[...], a1_ref[...]], axis=1)
    feat = _leaky(h * nd + b_ref[...])
    feat_ref[...] = feat
    if not last:
        ns = _inv_sqrt_deg(dego_ref[...])
        hn = lax.dot_general(feat * ns, Wn_ref[...], (((1,), (0,)), ((), ())),
                             precision=_HIGH)
        h0_ref[...] = hn[:, :128]
        h1_ref[...] = hn[:, 128:]


def _layer_post(agg0, agg1, W, b, degi, dego, Wn, first, last):
    rb = 1000
    b2 = b.reshape(1, 256)
    in_specs = [
        pl.BlockSpec((rb, 128), lambda i: (i, 0)),
        pl.BlockSpec((rb, 128), lambda i: (i, 0)),
        pl.BlockSpec(W.shape, lambda i: (0, 0)),
        pl.BlockSpec((1, 256), lambda i: (0, 0)),
        pl.BlockSpec((rb, 1), lambda i: (i, 0)),
        pl.BlockSpec((rb, 1), lambda i: (i, 0)),
        pl.BlockSpec(Wn.shape, lambda i: (0, 0)),
    ]
    out_specs = [
        pl.BlockSpec((rb, 256), lambda i: (i, 0)),
        pl.BlockSpec((rb, 128), lambda i: (i, 0)),
        pl.BlockSpec((rb, 128), lambda i: (i, 0)),
    ]
    out_shape = [_f32((N, 256)), _f32((N, 128)), _f32((N, 128))]
    return pl.pallas_call(
        functools.partial(_layer_kernel, first, last),
        grid=(N // rb,),
        in_specs=in_specs,
        out_specs=out_specs,
        out_shape=out_shape,
    )(agg0, agg1, W, b2, degi, dego, Wn)


def _set2set_kernel(feat_ref, Wih0T_ref, Whh0T_ref, b0_ref, Wih1T_ref,
                    Whh1T_ref, b1_ref, q_ref):
    d = 256
    feat = feat_ref[...]
    feat_bf = feat.astype(jnp.bfloat16)

    def cell(xv, h, c, WihT, WhhT, b):
        g = (lax.dot_general(xv, WihT, (((1,), (0,)), ((), ())), precision=_HIGH)
             + lax.dot_general(h, WhhT, (((1,), (0,)), ((), ())), precision=_HIGH)
             + b)
        i = jax.nn.sigmoid(g[:, :d])
        f = jax.nn.sigmoid(g[:, d:2 * d])
        gg = jnp.tanh(g[:, 2 * d:3 * d])
        o = jax.nn.sigmoid(g[:, 3 * d:])
        c = f * c + i * gg
        h = o * jnp.tanh(c)
        return h, c

    q_star = jnp.zeros((1, 2 * d), jnp.float32)
    h0 = jnp.zeros((1, d), jnp.float32)
    c0 = jnp.zeros((1, d), jnp.float32)
    h1 = jnp.zeros((1, d), jnp.float32)
    c1 = jnp.zeros((1, d), jnp.float32)
    for _ in range(N_STEP):
        h0, c0 = cell(q_star, h0, c0, Wih0T_ref[...], Whh0T_ref[...], b0_ref[...])
        h1, c1 = cell(h0, h1, c1, Wih1T_ref[...], Whh1T_ref[...], b1_ref[...])
        fb = feat_bf.astype(jnp.float32)
        e = jnp.sum(fb * h1, axis=1, keepdims=True)
        m = jnp.max(e)
        a = jnp.exp(e - m)
        a = a / jnp.sum(a)
        r = jnp.sum(a * fb, axis=0, keepdims=True)
        q_star = jnp.concatenate([h1, r], axis=1)
    q_ref[...] = q_star


def _set2set(feat, Wih0, Whh0, bih0, bhh0, Wih1, Whh1, bih1, bhh1):
    b0 = (bih0 + bhh0).reshape(1, 1024)
    b1 = (bih1 + bhh1).reshape(1, 1024)
    return pl.pallas_call(
        _set2set_kernel,
        out_shape=_f32((1, 512)),
    )(feat, Wih0.T, Whh0.T, b0, Wih1.T, Whh1.T, b1)


def _final_kernel(q0_ref, q1_ref, q2_ref, d1W_ref, d1b_ref, d2W_ref, d2b_ref,
                  o_ref):
    merged = jnp.concatenate([q0_ref[...], q1_ref[...], q2_ref[...]], axis=1)
    d1 = lax.dot_general(merged, d1W_ref[...], (((1,), (0,)), ((), ())),
                         precision=_HIGH) + d1b_ref[...]
    d2 = lax.dot_general(d1, d2W_ref[...], (((1,), (0,)), ((), ())),
                         precision=_HIGH) + d2b_ref[...]
    o_ref[...] = jax.nn.sigmoid(d2)


def _final(q0, q1, q2, d1W, d1b, d2W, d2b):
    return pl.pallas_call(
        _final_kernel,
        out_shape=_f32((1, 2)),
    )(q0, q1, q2, d1W, d1b.reshape(1, 128), d2W, d2b.reshape(1, 2))


def kernel(x, edge_index, gcW0, gcb0, gcW1, gcb1, gcW2, gcb2,
           s0Wih0, s0Whh0, s0bih0, s0bhh0, s0Wih1, s0Whh1, s0bih1, s0bhh1,
           s1Wih0, s1Whh0, s1bih0, s1bhh0, s1Wih1, s1Whh1, s1bih1, s1bhh1,
           s2Wih0, s2Whh0, s2bih0, s2bhh0, s2Wih1, s2Whh1, s2bih1, s2bhh1,
           d1W, d1b, d2W, d2b):
    sg16, dg16, sd16, sg32, dg32 = _chunk_edges(edge_index[0], edge_index[1])
    iota128 = jnp.arange(128, dtype=jnp.int32).reshape(1, 128)
    z128 = jnp.zeros((NPAD, 128), jnp.float32)

    dego128, degi128 = _sc_degrees(sd16.reshape(-1), dg16.reshape(-1),
                                   iota128, z128)
    dego = dego128.reshape(-1)[:N].reshape(N, 1)
    degi = degi128.reshape(-1)[:N].reshape(N, 1)

    xn = _norm_x(x, dego)
    aggx0, aggx1 = _sc_aggregate_l0(xn, sg32, dg32, z128)
    feat0, h1a, h1b = _layer_post(aggx0, aggx1, gcW0, gcb0, degi, dego,
                                  gcW1, first=True, last=False)
    q0 = _set2set(feat0, s0Wih0, s0Whh0, s0bih0, s0bhh0,
                  s0Wih1, s0Whh1, s0bih1, s0bhh1)

    agg10, agg11 = _sc_aggregate(h1a, h1b, sg16, dg16, z128)
    feat1, h2a, h2b = _layer_post(agg10, agg11, gcW1, gcb1, degi, dego,
                                  gcW2, first=False, last=False)
    q1 = _set2set(feat1, s1Wih0, s1Whh0, s1bih0, s1bhh0,
                  s1Wih1, s1Whh1, s1bih1, s1bhh1)

    agg20, agg21 = _sc_aggregate(h2a, h2b, sg16, dg16, z128)
    feat2, _, _ = _layer_post(agg20, agg21, gcW2, gcb2, degi, dego,
                              gcW2, first=False, last=True)
    q2 = _set2set(feat2, s2Wih0, s2Whh0, s2bih0, s2bhh0,
                  s2Wih1, s2Whh1, s2bih1, s2bhh1)

    return _final(q0, q1, q2, d1W, d1b, d2W, d2b)


# speedup vs baseline: 1.0240x; 1.0240x over previous
import dataclasses
import functools

import jax
import jax.numpy as jnp
from jax import lax
from jax.experimental import pallas as pl
from jax.experimental.pallas import tpu as pltpu
from jax.experimental.pallas import tpu_sc as plsc

N = 10000
E = 320000
NS = 16
K = 128
C16 = 160
C32 = 80
CB = 32
CB0 = 40
NPAD = N + 16
STRIPE = 624
TAIL_A = NPAD - NS * STRIPE
TAIL_O = N - NS * STRIPE
N_STEP = 10

_mesh = plsc.VectorSubcoreMesh(core_axis_name="c", subcore_axis_name="s")
_HIGH = jax.lax.Precision.HIGHEST


def _f32(shape):
    return jax.ShapeDtypeStruct(shape, jnp.float32)


def _chunked(idx, workers, chunks, pad_vals):
    per = E // workers
    padw = chunks * K - per
    return jnp.concatenate(
        [idx.reshape(workers, per), jnp.broadcast_to(pad_vals[:padw], (workers, padw))],
        axis=1,
    ).reshape(workers * chunks, K)


def _chunk_edges(src, dst):
    pad_real = (jnp.arange(512, dtype=jnp.int32) * 37) % N
    pad_dummy = N + (jnp.arange(512, dtype=jnp.int32) % 16)
    sg16 = _chunked(src, NS, C16, pad_real)
    dg16 = _chunked(dst, NS, C16, pad_dummy)
    sd16 = _chunked(src, NS, C16, pad_dummy)
    sg32 = _chunked(src, 2 * NS, C32, pad_real)
    dg32 = _chunked(dst, 2 * NS, C32, pad_dummy)
    return sg16, dg16, sd16, sg32, dg32


def _agg_pipeline(h_h, sg_h, dg_h, acc, sidx_v, didx_v, rows_a, rows_b,
                  sem_a, sem_b, base0, nblk, cb):
    @pl.loop(0, nblk)
    def _(b):
        base = base0 + b * cb
        pltpu.sync_copy(sg_h.at[pl.ds(base, cb)], sidx_v)
        pltpu.sync_copy(dg_h.at[pl.ds(base, cb)], didx_v)
        pltpu.async_copy(h_h.at[sidx_v.at[0]], rows_a, sem_a)

        @pl.loop(0, cb, step=2)
        def _(j):
            pltpu.make_async_copy(h_h.at[sidx_v.at[j]], rows_a, sem_a).wait()
            pltpu.async_copy(h_h.at[sidx_v.at[j + 1]], rows_b, sem_b)
            pltpu.sync_copy(rows_a, acc.at[didx_v.at[j]], add=True)
            pltpu.make_async_copy(h_h.at[sidx_v.at[j + 1]], rows_b, sem_b).wait()

            @pl.when(j + 2 < cb)
            def _():
                pltpu.async_copy(h_h.at[sidx_v.at[j + 2]], rows_a, sem_a)

            pltpu.sync_copy(rows_b, acc.at[didx_v.at[j + 1]], add=True)

EWP = C16 * K
DH = 128

_sc_cp = pltpu.CompilerParams()
if "needs_layout_passes" in pltpu.CompilerParams.__dataclass_fields__:
    _sc_cp = dataclasses.replace(_sc_cp, needs_layout_passes=False)


@functools.partial(
    pl.kernel,
    out_type=[_f32((DH, 128)), _f32((DH, 128))],
    mesh=_mesh,
    compiler_params=_sc_cp,
    scratch_types=[
        pltpu.VMEM((EWP,), jnp.int32),
        pltpu.VMEM((DH, 128), jnp.float32),
        pltpu.VMEM((1, 128), jnp.int32),
        pltpu.VMEM_SHARED((DH, 128), jnp.float32),
    ],
)
def _sc_degrees(sdf_h, dgf_h, iota_h, zeros_h, dego_h, degi_h,
                idx_v, hist, idr_v, acc):
    cid = lax.axis_index("c")
    wid = lax.axis_index("s")
    pltpu.sync_copy(iota_h, idr_v)
    pltpu.sync_copy(zeros_h.at[pl.ds(0, DH)], hist)

    @pl.when(wid == 0)
    def _():
        pltpu.sync_copy(zeros_h.at[pl.ds(0, DH)], acc)

    ones16 = jnp.full((16,), 1.0, jnp.float32)

    def run(idxf_h, out_h):
        pltpu.sync_copy(idxf_h.at[pl.ds(wid * EWP, EWP)], idx_v)
        plsc.subcore_barrier()

        @pl.loop(0, EWP, step=16)
        def _(i):
            v = idx_v[pl.ds(i, 16)]
            r = lax.shift_right_logical(v, 7)
            c = lax.bitwise_and(v, 127)
            plsc.addupdate_scatter(hist, [r, c], ones16)

        pltpu.sync_copy(hist, acc.at[idr_v.at[0]], add=True)
        plsc.subcore_barrier()

        @pl.when(wid == 0)
        def _():
            pltpu.sync_copy(acc, out_h)

    @pl.when(cid == 0)
    def _():
        run(sdf_h, dego_h)

    @pl.when(cid == 1)
    def _():
        run(dgf_h, degi_h)


@functools.partial(
    pl.kernel,
    out_type=[_f32((N, 128)), _f32((N, 128))],
    mesh=_mesh,
    scratch_types=[
        pltpu.VMEM((CB0, K), jnp.int32),
        pltpu.VMEM((CB0, K), jnp.int32),
        pltpu.VMEM((K, 128), jnp.float32),
        pltpu.VMEM((K, 128), jnp.float32),
        pltpu.VMEM_SHARED((NPAD, 128), jnp.float32),
        pltpu.SemaphoreType.DMA,
        pltpu.SemaphoreType.DMA,
    ],
)
def _sc_aggregate_l0(xn_h, sg32_h, dg32_h, zeros_h, out0_h, out1_h,
                     sidx_v, didx_v, rows_a, rows_b, acc, sem_a, sem_b):
    cid = lax.axis_index("c")
    wid = lax.axis_index("s")
    gwid = cid * NS + wid
    pltpu.sync_copy(zeros_h.at[pl.ds(wid * STRIPE, STRIPE)],
                    acc.at[pl.ds(wid * STRIPE, STRIPE)])

    @pl.when(wid == 0)
    def _():
        pltpu.sync_copy(zeros_h.at[pl.ds(NS * STRIPE, TAIL_A)],
                        acc.at[pl.ds(NS * STRIPE, TAIL_A)])

    plsc.subcore_barrier()

    _agg_pipeline(xn_h, sg32_h, dg32_h, acc, sidx_v, didx_v, rows_a, rows_b,
                  sem_a, sem_b, gwid * C32, C32 // CB0, CB0)

    plsc.subcore_barrier()

    def flush(out_h):
        pltpu.sync_copy(acc.at[pl.ds(wid * STRIPE, STRIPE)],
                        out_h.at[pl.ds(wid * STRIPE, STRIPE)])

        @pl.when(wid == 0)
        def _():
            pltpu.sync_copy(acc.at[pl.ds(NS * STRIPE, TAIL_O)],
                            out_h.at[pl.ds(NS * STRIPE, TAIL_O)])

    @pl.when(cid == 0)
    def _():
        flush(out0_h)

    @pl.when(cid == 1)
    def _():
        flush(out1_h)


@functools.partial(
    pl.kernel,
    out_type=[_f32((N, 128)), _f32((N, 128))],
    mesh=_mesh,
    scratch_types=[
        pltpu.VMEM((CB, K), jnp.int32),
        pltpu.VMEM((CB, K), jnp.int32),
        pltpu.VMEM((K, 128), jnp.float32),
        pltpu.VMEM((K, 128), jnp.float32),
        pltpu.VMEM_SHARED((NPAD, 128), jnp.float32),
        pltpu.SemaphoreType.DMA,
        pltpu.SemaphoreType.DMA,
    ],
)
def _sc_aggregate(h0_h, h1_h, sg16_h, dg16_h, zeros_h, out0_h, out1_h,
                  sidx_v, didx_v, rows_a, rows_b, acc, sem_a, sem_b):
    cid = lax.axis_index("c")
    wid = lax.axis_index("s")
    pltpu.sync_copy(zeros_h.at[pl.ds(wid * STRIPE, STRIPE)],
                    acc.at[pl.ds(wid * STRIPE, STRIPE)])

    @pl.when(wid == 0)
    def _():
        pltpu.sync_copy(zeros_h.at[pl.ds(NS * STRIPE, TAIL_A)],
                        acc.at[pl.ds(NS * STRIPE, TAIL_A)])

    plsc.subcore_barrier()

    def run(h_h, out_h):
        _agg_pipeline(h_h, sg16_h, dg16_h, acc, sidx_v, didx_v, rows_a,
                      rows_b, sem_a, sem_b, wid * C16, C16 // CB, CB)
        plsc.subcore_barrier()
        pltpu.sync_copy(acc.at[pl.ds(wid * STRIPE, STRIPE)],
                        out_h.at[pl.ds(wid * STRIPE, STRIPE)])

        @pl.when(wid == 0)
        def _():
            pltpu.sync_copy(acc.at[pl.ds(NS * STRIPE, TAIL_O)],
                            out_h.at[pl.ds(NS * STRIPE, TAIL_O)])

    @pl.when(cid == 0)
    def _():
        run(h0_h, out0_h)

    @pl.when(cid == 1)
    def _():
        run(h1_h, out1_h)


def _inv_sqrt_deg(deg_col):
    return jnp.where(deg_col > 0.0, lax.rsqrt(jnp.maximum(deg_col, 1e-30)), 0.0)


def _leaky(h):
    return jnp.where(h > 0.0, h, 0.01 * h)


def _norm_x_kernel(x_ref, dego_ref, xn_ref):
    ns = _inv_sqrt_deg(dego_ref[...])
    xn_ref[...] = x_ref[...] * ns


def _norm_x(x, dego):
    rb = 1000
    return pl.pallas_call(
        _norm_x_kernel,
        grid=(N // rb,),
        in_specs=[
            pl.BlockSpec((rb, 128), lambda i: (i, 0)),
            pl.BlockSpec((rb, 1), lambda i: (i, 0)),
        ],
        out_specs=pl.BlockSpec((rb, 128), lambda i: (i, 0)),
        out_shape=_f32((N, 128)),
    )(x, dego)


def _layer_kernel(first, last, a0_ref, a1_ref, W_ref, b_ref, degi_ref,
                  dego_ref, Wn_ref, feat_ref, h0_ref, h1_ref):
    nd = _inv_sqrt_deg(degi_ref[...])
    if first:
        aggv = a0_ref[...] + a1_ref[...]
        h = lax.dot_general(aggv, W_ref[...], (((1,), (0,)), ((), ())),
                            precision=_HIGH)
    else:
        h = jnp.concatenate([a0_ref[...], a1_ref[...]], axis=1)
    feat = _leaky(h * nd + b_ref[...])
    feat_ref[...] = feat
    if not last:
        ns = _inv_sqrt_deg(dego_ref[...])
        hn = lax.dot_general(feat * ns, Wn_ref[...], (((1,), (0,)), ((), ())),
                             precision=_HIGH)
        h0_ref[...] = hn[:, :128]
        h1_ref[...] = hn[:, 128:]


def _layer_post(agg0, agg1, W, b, degi, dego, Wn, first, last):
    rb = 1000
    b2 = b.reshape(1, 256)
    in_specs = [
        pl.BlockSpec((rb, 128), lambda i: (i, 0)),
        pl.BlockSpec((rb, 128), lambda i: (i, 0)),
        pl.BlockSpec(W.shape, lambda i: (0, 0)),
        pl.BlockSpec((1, 256), lambda i: (0, 0)),
        pl.BlockSpec((rb, 1), lambda i: (i, 0)),
        pl.BlockSpec((rb, 1), lambda i: (i, 0)),
        pl.BlockSpec(Wn.shape, lambda i: (0, 0)),
    ]
    out_specs = [
        pl.BlockSpec((rb, 256), lambda i: (i, 0)),
        pl.BlockSpec((rb, 128), lambda i: (i, 0)),
        pl.BlockSpec((rb, 128), lambda i: (i, 0)),
    ]
    out_shape = [_f32((N, 256)), _f32((N, 128)), _f32((N, 128))]
    return pl.pallas_call(
        functools.partial(_layer_kernel, first, last),
        grid=(N // rb,),
        in_specs=in_specs,
        out_specs=out_specs,
        out_shape=out_shape,
    )(agg0, agg1, W, b2, degi, dego, Wn)


def _set2set_body(feat, Wih0T, Whh0T, b0, Wih1T, Whh1T, b1):
    d = 256

    def cell(xv, h, c, WihT, WhhT, b):
        g = (lax.dot_general(xv, WihT, (((1,), (0,)), ((), ())), precision=_HIGH)
             + lax.dot_general(h, WhhT, (((1,), (0,)), ((), ())), precision=_HIGH)
             + b)
        i = jax.nn.sigmoid(g[:, :d])
        f = jax.nn.sigmoid(g[:, d:2 * d])
        gg = jnp.tanh(g[:, 2 * d:3 * d])
        o = jax.nn.sigmoid(g[:, 3 * d:])
        c = f * c + i * gg
        h = o * jnp.tanh(c)
        return h, c

    q_star = jnp.zeros((1, 2 * d), jnp.float32)
    h0 = jnp.zeros((1, d), jnp.float32)
    c0 = jnp.zeros((1, d), jnp.float32)
    h1 = jnp.zeros((1, d), jnp.float32)
    c1 = jnp.zeros((1, d), jnp.float32)
    for _ in range(N_STEP):
        h0, c0 = cell(q_star, h0, c0, Wih0T, Whh0T, b0)
        h1, c1 = cell(h0, h1, c1, Wih1T, Whh1T, b1)
        e = jnp.sum(feat * h1, axis=1, keepdims=True)
        m = jnp.max(e)
        a = jnp.exp(e - m)
        a = a / jnp.sum(a)
        r = jnp.sum(a * feat, axis=0, keepdims=True)
        q_star = jnp.concatenate([h1, r], axis=1)
    return q_star


def _set2set_kernel(feat_ref, Wih0T_ref, Whh0T_ref, b0_ref, Wih1T_ref,
                    Whh1T_ref, b1_ref, q_ref):
    q_ref[...] = _set2set_body(feat_ref[...], Wih0T_ref[...], Whh0T_ref[...],
                               b0_ref[...], Wih1T_ref[...], Whh1T_ref[...],
                               b1_ref[...])


def _set2set(feat, Wih0, Whh0, bih0, bhh0, Wih1, Whh1, bih1, bhh1):
    b0 = (bih0 + bhh0).reshape(1, 1024)
    b1 = (bih1 + bhh1).reshape(1, 1024)
    return pl.pallas_call(
        _set2set_kernel,
        out_shape=_f32((1, 512)),
    )(feat, Wih0.T, Whh0.T, b0, Wih1.T, Whh1.T, b1)


def _tail_kernel(a0_ref, a1_ref, degi_ref, b2_ref, Wih0T_ref, Whh0T_ref,
                 b0_ref, Wih1T_ref, Whh1T_ref, b1_ref, q0_ref, q1_ref,
                 d1W_ref, d1b_ref, d2W_ref, d2b_ref, o_ref):
    nd = _inv_sqrt_deg(degi_ref[...])
    h = jnp.concatenate([a0_ref[...], a1_ref[...]], axis=1)
    feat = _leaky(h * nd + b2_ref[...])
    q2 = _set2set_body(feat, Wih0T_ref[...], Whh0T_ref[...], b0_ref[...],
                       Wih1T_ref[...], Whh1T_ref[...], b1_ref[...])
    merged = jnp.concatenate([q0_ref[...], q1_ref[...], q2], axis=1)
    d1 = lax.dot_general(merged, d1W_ref[...], (((1,), (0,)), ((), ())),
                         precision=_HIGH) + d1b_ref[...]
    d2 = lax.dot_general(d1, d2W_ref[...], (((1,), (0,)), ((), ())),
                         precision=_HIGH) + d2b_ref[...]
    o_ref[...] = jax.nn.sigmoid(d2)


def _tail(agg0, agg1, degi, b2, Wih0, Whh0, bih0, bhh0, Wih1, Whh1, bih1,
          bhh1, q0, q1, d1W, d1b, d2W, d2b):
    b0 = (bih0 + bhh0).reshape(1, 1024)
    b1 = (bih1 + bhh1).reshape(1, 1024)
    return pl.pallas_call(
        _tail_kernel,
        out_shape=_f32((1, 2)),
    )(agg0, agg1, degi, b2.reshape(1, 256), Wih0.T, Whh0.T, b0, Wih1.T,
      Whh1.T, b1, q0, q1, d1W, d1b.reshape(1, 128), d2W, d2b.reshape(1, 2))


def kernel(x, edge_index, gcW0, gcb0, gcW1, gcb1, gcW2, gcb2,
           s0Wih0, s0Whh0, s0bih0, s0bhh0, s0Wih1, s0Whh1, s0bih1, s0bhh1,
           s1Wih0, s1Whh0, s1bih0, s1bhh0, s1Wih1, s1Whh1, s1bih1, s1bhh1,
           s2Wih0, s2Whh0, s2bih0, s2bhh0, s2Wih1, s2Whh1, s2bih1, s2bhh1,
           d1W, d1b, d2W, d2b):
    sg16, dg16, sd16, sg32, dg32 = _chunk_edges(edge_index[0], edge_index[1])
    iota128 = jnp.arange(128, dtype=jnp.int32).reshape(1, 128)
    z128 = jnp.zeros((NPAD, 128), jnp.float32)

    dego128, degi128 = _sc_degrees(sd16.reshape(-1), dg16.reshape(-1),
                                   iota128, z128)
    dego = dego128.reshape(-1)[:N].reshape(N, 1)
    degi = degi128.reshape(-1)[:N].reshape(N, 1)

    xn = _norm_x(x, dego)
    aggx0, aggx1 = _sc_aggregate_l0(xn, sg32, dg32, z128)
    feat0, h1a, h1b = _layer_post(aggx0, aggx1, gcW0, gcb0, degi, dego,
                                  gcW1, first=True, last=False)
    q0 = _set2set(feat0, s0Wih0, s0Whh0, s0bih0, s0bhh0,
                  s0Wih1, s0Whh1, s0bih1, s0bhh1)

    agg10, agg11 = _sc_aggregate(h1a, h1b, sg16, dg16, z128)
    feat1, h2a, h2b = _layer_post(agg10, agg11, gcW1, gcb1, degi, dego,
                                  gcW2, first=False, last=False)
    q1 = _set2set(feat1, s1Wih0, s1Whh0, s1bih0, s1bhh0,
                  s1Wih1, s1Whh1, s1bih1, s1bhh1)

    agg20, agg21 = _sc_aggregate(h2a, h2b, sg16, dg16, z128)
    return _tail(agg20, agg21, degi, gcb2, s2Wih0, s2Whh0, s2bih0, s2bhh0,
                 s2Wih1, s2Whh1, s2bih1, s2bhh1, q0, q1, d1W, d1b, d2W, d2b)
